# baseline (speedup 1.0000x reference)
import jax
import jax.numpy as jnp
import numpy as np
from jax import lax
from jax.experimental import pallas as pl
from jax.experimental.pallas import tpu as pltpu
from jax.experimental.pallas import tpu_sc as plsc

N = 10000
E = 320000
F = 128
G = 4
FG = F // G
GW = 3 * FG
NW = 32
EPW = E // NW
C = 40
NCHUNK = EPW // C
NPAD = 10240
RPS = NPAD // 16

_P32 = np.arange(32).reshape(2, 16).T.reshape(-1)


_BN = 2000


def _phi_body(x_ref, w1_ref, b1_ref, w2_0, w2_1, w2_2, w2_3,
              b2_0, b2_1, b2_2, b2_3, o0, o1, o2, o3):
    x = x_ref[...]
    h = jnp.dot(x, w1_ref[...], preferred_element_type=jnp.float32) + b1_ref[...]
    h = h * (1.0 / (1.0 + jnp.exp(-h)))
    for w2g, b2g, og in ((w2_0, b2_0, o0), (w2_1, b2_1, o1),
                         (w2_2, b2_2, o2), (w2_3, b2_3, o3)):
        og[...] = (jnp.dot(h, w2g[...], preferred_element_type=jnp.float32)
                   + b2g[...]).astype(jnp.bfloat16)


def _phi_tables(scalar_features, W1, b1, w2g, b2g):
    full = lambda shape: pl.BlockSpec(shape, lambda i: (0,) * len(shape))
    return pl.pallas_call(
        _phi_body,
        grid=(N // _BN,),
        in_specs=[pl.BlockSpec((_BN, F), lambda i: (i, 0)),
                  full((F, F)), full((1, F)),
                  full((F, GW)), full((F, GW)), full((F, GW)), full((F, GW)),
                  full((1, GW)), full((1, GW)), full((1, GW)), full((1, GW))],
        out_specs=[pl.BlockSpec((_BN, GW), lambda i: (i, 0))] * G,
        out_shape=[jax.ShapeDtypeStruct((N, GW), jnp.bfloat16)] * G,
    )(scalar_features, W1, b1.reshape(1, F), *w2g, *b2g)


_BE = 2000
_RP = 24


def _wf_body(rbf_ref, cut_ref, dir_ref, w3_0, w3_1, w3_2, w3_3,
             b3_0, b3_1, b3_2, b3_3, o0, o1, o2, o3):
    rbf = rbf_ref[...]
    cut = cut_ref[...]
    d = dir_ref[...]
    z1 = jnp.zeros((_BE, 1), jnp.float32)
    z27 = jnp.zeros((_BE, 27), jnp.float32)
    dblk = jnp.concatenate([d[:, 0:1], z1, d[:, 1:2], z1, d[:, 2:3], z27],
                           axis=1)
    for w3g, b3g, og in ((w3_0, b3_0, o0), (w3_1, b3_1, o1),
                         (w3_2, b3_2, o2), (w3_3, b3_3, o3)):
        wf = (jnp.dot(rbf, w3g[...], preferred_element_type=jnp.float32)
              + b3g[...]) * cut
        og[...] = jnp.concatenate([wf, dblk], axis=1).astype(jnp.bfloat16)


def _wf_tables(rbf_pad, cut2, rel_dir, w3g, b3g):
    full = lambda shape: pl.BlockSpec(shape, lambda i: (0,) * len(shape))
    return pl.pallas_call(
        _wf_body,
        grid=(E // _BE,),
        in_specs=[pl.BlockSpec((_BE, _RP), lambda i: (i, 0)),
                  pl.BlockSpec((_BE, 1), lambda i: (i, 0)),
                  pl.BlockSpec((_BE, 3), lambda i: (i, 0)),
                  full((_RP, GW)), full((_RP, GW)), full((_RP, GW)), full((_RP, GW)),
                  full((1, GW)), full((1, GW)), full((1, GW)), full((1, GW))],
        out_specs=[pl.BlockSpec((_BE, F), lambda i: (i, 0))] * G,
        out_shape=[jax.ShapeDtypeStruct((E, F), jnp.bfloat16)] * G,
    )(rbf_pad, cut2, rel_dir, *w3g, *b3g)


def _sc_body(idx_i, idx_j, wf0, wf1, wf2, wf3,
             ph0, ph1, ph2, ph3, vf0, vf1, vf2, vf3, zacc, zcnt,
             part_out, cnt_out,
             ij0, ij1, ij2, ij3, ii0, ii1, ii2, ii3,
             wfb0, wfb1, wfb2, wfb3,
             phb0, phb1, phb2, phb3, vfb0, vfb1, vfb2, vfb3,
             ctb0, ctb1, ones_v,
             acc, cnts,
             semX0, semX1, semX2, semX3, semP0, semP1, semP2, semP3,
             semV0, semV1, semV2, semV3, semS0, semS1,
             semI0, semI1, semI2, semI3, semC0, semC1):
    cid = lax.axis_index("c")
    sid = lax.axis_index("s")
    wid = cid * 16 + sid
    ebase = wid * EPW
    r0 = sid * RPS

    ij = (ij0, ij1, ij2, ij3)
    ii = (ii0, ii1, ii2, ii3)
    wfb_ = (wfb0, wfb1, wfb2, wfb3)
    phb = (phb0, phb1, phb2, phb3)
    vfb = (vfb0, vfb1, vfb2, vfb3)
    ctb = (ctb0, ctb1)
    semX = (semX0, semX1, semX2, semX3)
    semP = (semP0, semP1, semP2, semP3)
    semV = (semV0, semV1, semV2, semV3)
    semS = (semS0, semS1)
    semI = (semI0, semI1, semI2, semI3)
    semC = (semC0, semC1)

    def fill_ones(r, u):
        ones_v[r, :] = jnp.ones((16,), jnp.float32)
        return u
    lax.fori_loop(0, C, fill_ones, 0)

    NT = NCHUNK // 4

    for g, (wfg, phg, vfg) in enumerate(
            ((wf0, ph0, vf0), (wf1, ph1, vf1), (wf2, ph2, vf2), (wf3, ph3, vf3))):

        def eb_of(k):
            return ebase + k * C

        def issue_linX(k, s):
            eb = eb_of(k)
            pltpu.async_copy(idx_j.at[pl.ds(eb, C)], ij[s], semX[s])
            pltpu.async_copy(wfg.at[pl.ds(eb, C)], wfb_[s], semX[s])

        def sync_linX(k, s):
            eb = eb_of(k)
            pltpu.sync_copy(idx_j.at[pl.ds(eb, C)], ij[s])
            pltpu.sync_copy(wfg.at[pl.ds(eb, C)], wfb_[s])

        def wait_linX(k, s):
            eb = eb_of(k)
            pltpu.make_async_copy(idx_j.at[pl.ds(eb, C)], ij[s], semX[s]).wait()
            pltpu.make_async_copy(wfg.at[pl.ds(eb, C)], wfb_[s], semX[s]).wait()

        def issue_gathers(s):
            pltpu.async_copy(phg.at[ij[s]], phb[s], semP[s])
            pltpu.async_copy(vfg.at[ij[s]], vfb[s], semV[s])

        def wait_gathers(s):
            pltpu.make_async_copy(phg.at[ij[s]], phb[s], semP[s]).wait()
            pltpu.make_async_copy(vfg.at[ij[s]], vfb[s], semV[s]).wait()

        def wait_scatter(s, islot):
            pltpu.make_async_copy(ctb[s], acc.at[ii[islot]], semS[s]).wait()
            if g == 0:
                pltpu.make_async_copy(ones_v, cnts.at[ii[islot]],
                                      semC[s]).wait()

        def wait_ii(k, s):
            pltpu.make_async_copy(idx_i.at[pl.ds(eb_of(k), C)], ii[s],
                                  semI[s]).wait()

        def compute(s, cslot):
            phbS, wfbS, vfbS = phb[s], wfb_[s], vfb[s]
            ctbS = ctb[cslot]
            unp = lambda x: plsc.unpack(x, format=plsc.PackFormat.INTERLEAVED)

            def edge(e, u):
                dv = unp(wfbS[e, pl.ds(96, 32)])[0]
                dx = dv[0]
                dy = dv[1]
                dz = dv[2]
                phvv = unp(phbS[e, pl.ds(0, 32)])
                phss = unp(phbS[e, pl.ds(32, 32)])
                phvs = unp(phbS[e, pl.ds(64, 32)])
                wfvv = unp(wfbS[e, pl.ds(0, 32)])
                wfss = unp(wfbS[e, pl.ds(32, 32)])
                wfvs = unp(wfbS[e, pl.ds(64, 32)])
                vfx = unp(vfbS[e, pl.ds(0, 32)])
                vfy = unp(vfbS[e, pl.ds(32, 32)])
                vfz = unp(vfbS[e, pl.ds(64, 32)])
                for c in range(2):
                    sl = lambda a: pl.ds(16 * a + 16 * c, 16)
                    pvv = phvv[c] * wfvv[c]
                    pvs = phvs[c] * wfvs[c]
                    ctbS[e, sl(0)] = phss[c] * wfss[c]
                    ctbS[e, sl(2)] = vfx[c] * pvv + pvs * dx
                    ctbS[e, sl(4)] = vfy[c] * pvv + pvs * dy
                    ctbS[e, sl(6)] = vfz[c] * pvv + pvs * dz
                return u
            lax.fori_loop(0, C, edge, 0)

        pltpu.sync_copy(zacc.at[pl.ds(r0, RPS)], acc.at[pl.ds(r0, RPS)])
        if g == 0:
            pltpu.sync_copy(zcnt.at[pl.ds(r0, RPS)], cnts.at[pl.ds(r0, RPS)])
        plsc.subcore_barrier()

        sync_linX(0, 0)
        sync_linX(1, 1)
        sync_linX(2, 2)
        issue_linX(3, 3)
        pltpu.sync_copy(idx_i.at[pl.ds(eb_of(0), C)], ii[0])
        pltpu.sync_copy(idx_i.at[pl.ds(eb_of(1), C)], ii[1])
        issue_gathers(0)
        issue_gathers(1)
        issue_gathers(2)

        def quad(t, carry):
            for j in range(4):
                k = 4 * t + j
                p = j % 2
                wait_gathers(j)
                if j == 3:
                    @pl.when(t < NT - 1)
                    def _():
                        wait_linX(k + 3, 2)
                        issue_gathers(2)
                else:
                    wait_linX(k + 3, (j + 3) % 4)
                    issue_gathers((j + 3) % 4)
                if j < 2:
                    @pl.when(t >= 1)
                    def _():
                        wait_scatter(p, (j + 2) % 4)
                else:
                    wait_scatter(p, (j + 2) % 4)
                pltpu.async_copy(idx_i.at[pl.ds(eb_of(k + 2), C)],
                                 ii[(j + 2) % 4], semI[(j + 2) % 4])
                compute(j, p)
                if j < 2:
                    @pl.when(t >= 1)
                    def _():
                        wait_ii(k, j)
                else:
                    wait_ii(k, j)
                pltpu.async_copy(ctb[p], acc.at[ii[j]], semS[p], add=True)
                if g == 0:
                    pltpu.async_copy(ones_v, cnts.at[ii[j]], semC[p], add=True)
                if j < 2:
                    issue_linX(k + 4, j)
                else:
                    @pl.when(t < NT - 1)
                    def _():
                        issue_linX(k + 4, j)
            return carry
        lax.fori_loop(0, NT, quad, 0)

        ka, kb = NCHUNK - 2, NCHUNK - 1
        wait_gathers(0)
        wait_scatter(0, 2)
        compute(0, 0)
        wait_ii(ka, 0)
        pltpu.async_copy(ctb[0], acc.at[ii[0]], semS[0], add=True)
        if g == 0:
            pltpu.async_copy(ones_v, cnts.at[ii[0]], semC[0], add=True)
        wait_gathers(1)
        wait_scatter(1, 3)
        compute(1, 1)
        wait_ii(kb, 1)
        pltpu.sync_copy(ctb[1], acc.at[ii[1]], add=True)
        if g == 0:
            pltpu.sync_copy(ones_v, cnts.at[ii[1]], add=True)
        wait_scatter(0, 0)

        plsc.subcore_barrier()
        pltpu.sync_copy(acc.at[pl.ds(r0, RPS)],
                        part_out.at[cid, g, pl.ds(r0, RPS)])
        if g == 0:
            pltpu.sync_copy(cnts.at[pl.ds(r0, RPS)],
                            cnt_out.at[cid, pl.ds(r0, RPS)])
        plsc.subcore_barrier()


def _sc_partials(idx_i, idx_j, wfg, phg, vfg, zacc, zcnt):
    mesh = plsc.VectorSubcoreMesh(core_axis_name="c", subcore_axis_name="s")
    f = pl.kernel(
        _sc_body,
        mesh=mesh,
        compiler_params=pltpu.CompilerParams(use_tc_tiling_on_sc=False,
                                             needs_layout_passes=False),
        out_type=(jax.ShapeDtypeStruct((2, G, NPAD, F), jnp.float32),
                  jax.ShapeDtypeStruct((2, NPAD, 16), jnp.float32)),
        scratch_types=(
            [pltpu.VMEM((C,), jnp.int32)] * 8
            + [pltpu.VMEM((C, F), jnp.bfloat16)] * 4
            + [pltpu.VMEM((C, GW), jnp.bfloat16)] * 8
            + [pltpu.VMEM((C, F), jnp.float32)] * 2
            + [pltpu.VMEM((C, 16), jnp.float32)]
            + [pltpu.VMEM_SHARED((NPAD, F), jnp.float32),
               pltpu.VMEM_SHARED((NPAD, 16), jnp.float32)]
            + [pltpu.SemaphoreType.DMA] * 20
        ),
    )
    return f(idx_i, idx_j, *wfg, *phg, *vfg, zacc, zcnt)


_BD = 1000


def _comb_body(part_ref, cnt_ref, sf_ref, vft_ref, so_ref, vo_ref):
    p = part_ref[0] + part_ref[1]
    count = cnt_ref[0, :, 0:1] + cnt_ref[1, :, 0:1]
    inv = 1.0 / count
    so_ref[...] = sf_ref[...] + jnp.concatenate(
        [p[g, :, 0:FG] * inv for g in range(G)], axis=1)
    for d in range(3):
        vo_ref[d] = vft_ref[d] + jnp.concatenate(
            [p[g, :, FG + d * FG:2 * FG + d * FG] * inv for g in range(G)],
            axis=1)


def _combine(part, cnt, scalar_features, vft):
    return pl.pallas_call(
        _comb_body,
        grid=(N // _BD,),
        in_specs=[pl.BlockSpec((2, G, _BD, F), lambda i: (0, 0, i, 0)),
                  pl.BlockSpec((2, _BD, 16), lambda i: (0, i, 0)),
                  pl.BlockSpec((_BD, F), lambda i: (i, 0)),
                  pl.BlockSpec((3, _BD, F), lambda i: (0, i, 0))],
        out_specs=[pl.BlockSpec((_BD, F), lambda i: (i, 0)),
                   pl.BlockSpec((3, _BD, F), lambda i: (0, i, 0))],
        out_shape=[jax.ShapeDtypeStruct((N, F), jnp.float32),
                   jax.ShapeDtypeStruct((3, N, F), jnp.float32)],
    )(part, cnt, scalar_features, vft)


def _group_cols(w, b):
    ws, bs = [], []
    for g in range(G):
        idx = np.concatenate([sec * F + g * FG + _P32 for sec in range(3)])
        ws.append(w[:, idx])
        bs.append(b[idx].reshape(1, GW))
    return ws, bs


@jax.jit
def _run(idx_i, idx_j, rel_dir, rel_dist_cut, rbf_features,
         scalar_features, vector_features, W1, b1, W2, b2, W3, b3):
    w2g, b2g = _group_cols(W2, b2)
    w3g, b3g = _group_cols(W3, b3)
    phg = _phi_tables(scalar_features, W1, b1, w2g, b2g)

    R = rbf_features.shape[1]
    rbf_pad = jnp.pad(rbf_features, ((0, 0), (0, _RP - R)))
    w3g = [jnp.pad(w, ((0, _RP - R), (0, 0))) for w in w3g]
    wfg = _wf_tables(rbf_pad, rel_dist_cut.reshape(E, 1), rel_dir, w3g, b3g)

    vft = jnp.transpose(vector_features, (2, 0, 1))
    vfg = [jnp.concatenate([vft[d][:, g * FG + _P32] for d in range(3)],
                           axis=1).astype(jnp.bfloat16)
           for g in range(G)]

    zacc = jnp.zeros((NPAD, F), jnp.float32)
    zcnt = jnp.zeros((NPAD, 16), jnp.float32)
    part, cnt = _sc_partials(idx_i.astype(jnp.int32), idx_j.astype(jnp.int32),
                             wfg, phg, vfg, zacc, zcnt)

    so, vot = _combine(part, cnt, scalar_features, vft)
    return so, jnp.transpose(vot, (1, 2, 0))


def kernel(idx_i, idx_j, rel_dir, rel_dist_cut, rbf_features,
           scalar_features, vector_features, W1, b1, W2, b2, W3, b3):
    return _run(idx_i, idx_j, rel_dir, rel_dist_cut, rbf_features,
                scalar_features, vector_features, W1, b1, W2, b2, W3, b3)


# speedup vs baseline: 1.0071x; 1.0071x over previous
import jax
import jax.numpy as jnp
import numpy as np
from jax import lax
from jax.experimental import pallas as pl
from jax.experimental.pallas import tpu as pltpu
from jax.experimental.pallas import tpu_sc as plsc

N = 10000
E = 320000
F = 128
G = 4
FG = F // G
GW = 3 * FG
NW = 32
EPW = E // NW
C = 40
NCHUNK = EPW // C
NPAD = 10240
RPS = NPAD // 16

_P32 = np.arange(32).reshape(2, 16).T.reshape(-1)


_BN = 2000


def _phi_body(x_ref, w1_ref, b1_ref, w2_0, w2_1, w2_2, w2_3,
              b2_0, b2_1, b2_2, b2_3, o0, o1, o2, o3):
    x = x_ref[...]
    h = jnp.dot(x, w1_ref[...], preferred_element_type=jnp.float32) + b1_ref[...]
    h = h * (1.0 / (1.0 + jnp.exp(-h)))
    for w2g, b2g, og in ((w2_0, b2_0, o0), (w2_1, b2_1, o1),
                         (w2_2, b2_2, o2), (w2_3, b2_3, o3)):
        og[...] = (jnp.dot(h, w2g[...], preferred_element_type=jnp.float32)
                   + b2g[...]).astype(jnp.bfloat16)


def _phi_tables(scalar_features, W1, b1, w2g, b2g):
    full = lambda shape: pl.BlockSpec(shape, lambda i: (0,) * len(shape))
    return pl.pallas_call(
        _phi_body,
        grid=(N // _BN,),
        in_specs=[pl.BlockSpec((_BN, F), lambda i: (i, 0)),
                  full((F, F)), full((1, F)),
                  full((F, GW)), full((F, GW)), full((F, GW)), full((F, GW)),
                  full((1, GW)), full((1, GW)), full((1, GW)), full((1, GW))],
        out_specs=[pl.BlockSpec((_BN, GW), lambda i: (i, 0))] * G,
        out_shape=[jax.ShapeDtypeStruct((N, GW), jnp.bfloat16)] * G,
    )(scalar_features, W1, b1.reshape(1, F), *w2g, *b2g)


_BE = 2000
_RP = 24


def _wf_body(rbf_ref, cut_ref, dir_ref, w3_0, w3_1, w3_2, w3_3,
             b3_0, b3_1, b3_2, b3_3, o0, o1, o2, o3):
    rbf = rbf_ref[...]
    cut = cut_ref[...]
    d = dir_ref[...]
    z1 = jnp.zeros((_BE, 1), jnp.float32)
    z27 = jnp.zeros((_BE, 27), jnp.float32)
    dblk = jnp.concatenate([d[:, 0:1], z1, d[:, 1:2], z1, d[:, 2:3], z27],
                           axis=1)
    for w3g, b3g, og in ((w3_0, b3_0, o0), (w3_1, b3_1, o1),
                         (w3_2, b3_2, o2), (w3_3, b3_3, o3)):
        wf = (jnp.dot(rbf, w3g[...], preferred_element_type=jnp.float32)
              + b3g[...]) * cut
        row = jnp.concatenate([wf, dblk], axis=1).astype(jnp.bfloat16)
        og[...] = jnp.reshape(row, (_BE * F,))


def _wf_tables(rbf_pad, cut2, rel_dir, w3g, b3g):
    full = lambda shape: pl.BlockSpec(shape, lambda i: (0,) * len(shape))
    return pl.pallas_call(
        _wf_body,
        grid=(E // _BE,),
        in_specs=[pl.BlockSpec((_BE, _RP), lambda i: (i, 0)),
                  pl.BlockSpec((_BE, 1), lambda i: (i, 0)),
                  pl.BlockSpec((_BE, 3), lambda i: (i, 0)),
                  full((_RP, GW)), full((_RP, GW)), full((_RP, GW)), full((_RP, GW)),
                  full((1, GW)), full((1, GW)), full((1, GW)), full((1, GW))],
        out_specs=[pl.BlockSpec((_BE * F,), lambda i: (i,))] * G,
        out_shape=[jax.ShapeDtypeStruct((E * F,), jnp.bfloat16)] * G,
    )(rbf_pad, cut2, rel_dir, *w3g, *b3g)


def _sc_body(idx_i, idx_j, wf0, wf1, wf2, wf3,
             ph0, ph1, ph2, ph3, vf0, vf1, vf2, vf3, zacc, zcnt,
             part_out, cnt_out,
             ij0, ij1, ij2, ij3, ii0, ii1, ii2, ii3,
             wfb0, wfb1, wfb2, wfb3,
             phb0, phb1, phb2, phb3, vfb0, vfb1, vfb2, vfb3,
             ctb0, ctb1, ones_v,
             acc, cnts,
             semX0, semX1, semX2, semX3, semP0, semP1, semP2, semP3,
             semV0, semV1, semV2, semV3, semS0, semS1,
             semI0, semI1, semI2, semI3, semC0, semC1):
    cid = lax.axis_index("c")
    sid = lax.axis_index("s")
    wid = cid * 16 + sid
    ebase = wid * EPW
    r0 = sid * RPS

    ij = (ij0, ij1, ij2, ij3)
    ii = (ii0, ii1, ii2, ii3)
    wfb_ = (wfb0, wfb1, wfb2, wfb3)
    phb = (phb0, phb1, phb2, phb3)
    vfb = (vfb0, vfb1, vfb2, vfb3)
    ctb = (ctb0, ctb1)
    semX = (semX0, semX1, semX2, semX3)
    semP = (semP0, semP1, semP2, semP3)
    semV = (semV0, semV1, semV2, semV3)
    semS = (semS0, semS1)
    semI = (semI0, semI1, semI2, semI3)
    semC = (semC0, semC1)

    def fill_ones(r, u):
        ones_v[r, :] = jnp.ones((16,), jnp.float32)
        return u
    lax.fori_loop(0, C, fill_ones, 0)

    NT = NCHUNK // 4

    for g, (wfg, phg, vfg) in enumerate(
            ((wf0, ph0, vf0), (wf1, ph1, vf1), (wf2, ph2, vf2), (wf3, ph3, vf3))):

        def eb_of(k):
            return ebase + k * C

        def issue_linX(k, s):
            eb = eb_of(k)
            pltpu.async_copy(idx_j.at[pl.ds(eb, C)], ij[s], semX[s])
            pltpu.async_copy(wfg.at[pl.ds(eb * F, C * F)], wfb_[s], semX[s])

        def sync_linX(k, s):
            eb = eb_of(k)
            pltpu.sync_copy(idx_j.at[pl.ds(eb, C)], ij[s])
            pltpu.sync_copy(wfg.at[pl.ds(eb * F, C * F)], wfb_[s])

        def wait_linX(k, s):
            eb = eb_of(k)
            pltpu.make_async_copy(idx_j.at[pl.ds(eb, C)], ij[s], semX[s]).wait()
            pltpu.make_async_copy(wfg.at[pl.ds(eb * F, C * F)], wfb_[s],
                                  semX[s]).wait()

        def issue_gathers(s):
            pltpu.async_copy(phg.at[ij[s]], phb[s], semP[s])
            pltpu.async_copy(vfg.at[ij[s]], vfb[s], semV[s])

        def wait_gathers(s):
            pltpu.make_async_copy(phg.at[ij[s]], phb[s], semP[s]).wait()
            pltpu.make_async_copy(vfg.at[ij[s]], vfb[s], semV[s]).wait()

        def wait_scatter(s, islot):
            pltpu.make_async_copy(ctb[s], acc.at[ii[islot]], semS[s]).wait()
            if g == 0:
                pltpu.make_async_copy(ones_v, cnts.at[ii[islot]],
                                      semC[s]).wait()

        def wait_ii(k, s):
            pltpu.make_async_copy(idx_i.at[pl.ds(eb_of(k), C)], ii[s],
                                  semI[s]).wait()

        def compute(s, cslot):
            phbS, wfbS, vfbS = phb[s], wfb_[s], vfb[s]
            ctbS = ctb[cslot]
            unp = lambda x: plsc.unpack(x, format=plsc.PackFormat.INTERLEAVED)

            def edge(e, u):
                wb = e * F
                dv = unp(wfbS[pl.ds(wb + 96, 32)])[0]
                dx = dv[0]
                dy = dv[1]
                dz = dv[2]
                phvv = unp(phbS[e, pl.ds(0, 32)])
                phss = unp(phbS[e, pl.ds(32, 32)])
                phvs = unp(phbS[e, pl.ds(64, 32)])
                wfvv = unp(wfbS[pl.ds(wb, 32)])
                wfss = unp(wfbS[pl.ds(wb + 32, 32)])
                wfvs = unp(wfbS[pl.ds(wb + 64, 32)])
                vfx = unp(vfbS[e, pl.ds(0, 32)])
                vfy = unp(vfbS[e, pl.ds(32, 32)])
                vfz = unp(vfbS[e, pl.ds(64, 32)])
                for c in range(2):
                    sl = lambda a: pl.ds(16 * a + 16 * c, 16)
                    pvv = phvv[c] * wfvv[c]
                    pvs = phvs[c] * wfvs[c]
                    ctbS[e, sl(0)] = phss[c] * wfss[c]
                    ctbS[e, sl(2)] = vfx[c] * pvv + pvs * dx
                    ctbS[e, sl(4)] = vfy[c] * pvv + pvs * dy
                    ctbS[e, sl(6)] = vfz[c] * pvv + pvs * dz
                return u
            lax.fori_loop(0, C, edge, 0, unroll=2)

        pltpu.sync_copy(zacc.at[pl.ds(r0, RPS)], acc.at[pl.ds(r0, RPS)])
        if g == 0:
            pltpu.sync_copy(zcnt.at[pl.ds(r0, RPS)], cnts.at[pl.ds(r0, RPS)])
        plsc.subcore_barrier()

        sync_linX(0, 0)
        sync_linX(1, 1)
        sync_linX(2, 2)
        issue_linX(3, 3)
        pltpu.sync_copy(idx_i.at[pl.ds(eb_of(0), C)], ii[0])
        pltpu.sync_copy(idx_i.at[pl.ds(eb_of(1), C)], ii[1])
        issue_gathers(0)
        issue_gathers(1)
        issue_gathers(2)

        def quad(t, carry):
            for j in range(4):
                k = 4 * t + j
                p = j % 2
                wait_gathers(j)
                if j == 3:
                    @pl.when(t < NT - 1)
                    def _():
                        wait_linX(k + 3, 2)
                        issue_gathers(2)
                else:
                    wait_linX(k + 3, (j + 3) % 4)
                    issue_gathers((j + 3) % 4)
                if j < 2:
                    @pl.when(t >= 1)
                    def _():
                        wait_scatter(p, (j + 2) % 4)
                else:
                    wait_scatter(p, (j + 2) % 4)
                pltpu.async_copy(idx_i.at[pl.ds(eb_of(k + 2), C)],
                                 ii[(j + 2) % 4], semI[(j + 2) % 4])
                compute(j, p)
                if j < 2:
                    @pl.when(t >= 1)
                    def _():
                        wait_ii(k, j)
                else:
                    wait_ii(k, j)
                pltpu.async_copy(ctb[p], acc.at[ii[j]], semS[p], add=True)
                if g == 0:
                    pltpu.async_copy(ones_v, cnts.at[ii[j]], semC[p], add=True)
                if j < 2:
                    issue_linX(k + 4, j)
                else:
                    @pl.when(t < NT - 1)
                    def _():
                        issue_linX(k + 4, j)
            return carry
        lax.fori_loop(0, NT, quad, 0)

        ka, kb = NCHUNK - 2, NCHUNK - 1
        wait_gathers(0)
        wait_scatter(0, 2)
        compute(0, 0)
        wait_ii(ka, 0)
        pltpu.async_copy(ctb[0], acc.at[ii[0]], semS[0], add=True)
        if g == 0:
            pltpu.async_copy(ones_v, cnts.at[ii[0]], semC[0], add=True)
        wait_gathers(1)
        wait_scatter(1, 3)
        compute(1, 1)
        wait_ii(kb, 1)
        pltpu.sync_copy(ctb[1], acc.at[ii[1]], add=True)
        if g == 0:
            pltpu.sync_copy(ones_v, cnts.at[ii[1]], add=True)
        wait_scatter(0, 0)

        plsc.subcore_barrier()
        pltpu.sync_copy(acc.at[pl.ds(r0, RPS)],
                        part_out.at[cid, g, pl.ds(r0, RPS)])
        if g == 0:
            pltpu.sync_copy(cnts.at[pl.ds(r0, RPS)],
                            cnt_out.at[cid, pl.ds(r0, RPS)])
        plsc.subcore_barrier()


def _sc_partials(idx_i, idx_j, wfg, phg, vfg, zacc, zcnt):
    mesh = plsc.VectorSubcoreMesh(core_axis_name="c", subcore_axis_name="s")
    f = pl.kernel(
        _sc_body,
        mesh=mesh,
        compiler_params=pltpu.CompilerParams(use_tc_tiling_on_sc=False,
                                             needs_layout_passes=False),
        out_type=(jax.ShapeDtypeStruct((2, G, NPAD, F), jnp.float32),
                  jax.ShapeDtypeStruct((2, NPAD, 16), jnp.float32)),
        scratch_types=(
            [pltpu.VMEM((C,), jnp.int32)] * 8
            + [pltpu.VMEM((C * F,), jnp.bfloat16)] * 4
            + [pltpu.VMEM((C, GW), jnp.bfloat16)] * 8
            + [pltpu.VMEM((C, F), jnp.float32)] * 2
            + [pltpu.VMEM((C, 16), jnp.float32)]
            + [pltpu.VMEM_SHARED((NPAD, F), jnp.float32),
               pltpu.VMEM_SHARED((NPAD, 16), jnp.float32)]
            + [pltpu.SemaphoreType.DMA] * 20
        ),
    )
    return f(idx_i, idx_j, *wfg, *phg, *vfg, zacc, zcnt)


_BD = 1000


def _comb_body(part_ref, cnt_ref, sf_ref, vft_ref, so_ref, vo_ref):
    p = part_ref[0] + part_ref[1]
    count = cnt_ref[0, :, 0:1] + cnt_ref[1, :, 0:1]
    inv = 1.0 / count
    so_ref[...] = sf_ref[...] + jnp.concatenate(
        [p[g, :, 0:FG] * inv for g in range(G)], axis=1)
    for d in range(3):
        vo_ref[d] = vft_ref[d] + jnp.concatenate(
            [p[g, :, FG + d * FG:2 * FG + d * FG] * inv for g in range(G)],
            axis=1)


def _combine(part, cnt, scalar_features, vft):
    return pl.pallas_call(
        _comb_body,
        grid=(N // _BD,),
        in_specs=[pl.BlockSpec((2, G, _BD, F), lambda i: (0, 0, i, 0)),
                  pl.BlockSpec((2, _BD, 16), lambda i: (0, i, 0)),
                  pl.BlockSpec((_BD, F), lambda i: (i, 0)),
                  pl.BlockSpec((3, _BD, F), lambda i: (0, i, 0))],
        out_specs=[pl.BlockSpec((_BD, F), lambda i: (i, 0)),
                   pl.BlockSpec((3, _BD, F), lambda i: (0, i, 0))],
        out_shape=[jax.ShapeDtypeStruct((N, F), jnp.float32),
                   jax.ShapeDtypeStruct((3, N, F), jnp.float32)],
    )(part, cnt, scalar_features, vft)


def _group_cols(w, b):
    ws, bs = [], []
    for g in range(G):
        idx = np.concatenate([sec * F + g * FG + _P32 for sec in range(3)])
        ws.append(w[:, idx])
        bs.append(b[idx].reshape(1, GW))
    return ws, bs


@jax.jit
def _run(idx_i, idx_j, rel_dir, rel_dist_cut, rbf_features,
         scalar_features, vector_features, W1, b1, W2, b2, W3, b3):
    w2g, b2g = _group_cols(W2, b2)
    w3g, b3g = _group_cols(W3, b3)
    phg = _phi_tables(scalar_features, W1, b1, w2g, b2g)

    R = rbf_features.shape[1]
    rbf_pad = jnp.pad(rbf_features, ((0, 0), (0, _RP - R)))
    w3g = [jnp.pad(w, ((0, _RP - R), (0, 0))) for w in w3g]
    wfg = _wf_tables(rbf_pad, rel_dist_cut.reshape(E, 1), rel_dir, w3g, b3g)

    vft = jnp.transpose(vector_features, (2, 0, 1))
    vfg = [jnp.concatenate([vft[d][:, g * FG + _P32] for d in range(3)],
                           axis=1).astype(jnp.bfloat16)
           for g in range(G)]

    zacc = jnp.zeros((NPAD, F), jnp.float32)
    zcnt = jnp.zeros((NPAD, 16), jnp.float32)
    part, cnt = _sc_partials(idx_i.astype(jnp.int32), idx_j.astype(jnp.int32),
                             wfg, phg, vfg, zacc, zcnt)

    so, vot = _combine(part, cnt, scalar_features, vft)
    return so, jnp.transpose(vot, (1, 2, 0))


def kernel(idx_i, idx_j, rel_dir, rel_dist_cut, rbf_features,
           scalar_features, vector_features, W1, b1, W2, b2, W3, b3):
    return _run(idx_i, idx_j, rel_dir, rel_dist_cut, rbf_features,
                scalar_features, vector_features, W1, b1, W2, b2, W3, b3)
